# baseline (speedup 1.0000x reference)
import jax
import jax.numpy as jnp
from jax import lax
from jax.experimental import pallas as pl
from jax.experimental.pallas import tpu as pltpu
from jax.experimental.pallas import tpu_sc as plsc

N_NODES = 10000
N_EDGES = 320000
D = 128
NC, NS = 2, 16
NW = NC * NS
NPAD = 10240
RPT = NPAD // NS
K = 128
NCH = 80
NBLK = 10
EPW = NCH * K
E_PAD = NW * EPW

_MESH = plsc.VectorSubcoreMesh(
    core_axis_name="c", subcore_axis_name="s", num_cores=NC, num_subcores=NS)


def _segsum_body(r_hbm, srcg, dst2, out_hbm,
                 src_v, d0, d1, rows0, rows1, zbuf,
                 semA, semB, semD0, semD1, acc_sh):
    c = lax.axis_index("c")
    s = lax.axis_index("s")
    w = c * NS + s
    zv = jnp.zeros((16,), jnp.float32)
    for r in range(8):
        for k in range(D // 16):
            zbuf[r, pl.ds(k * 16, 16)] = zv

    def zrow(i, carry):
        pltpu.sync_copy(zbuf, acc_sh.at[pl.ds(s * RPT + i * 8, 8)])
        return carry

    lax.fori_loop(0, RPT // 8, zrow, 0)
    pltpu.sync_copy(srcg.at[w], src_v)
    plsc.subcore_barrier()

    dbase = w * NCH
    pltpu.async_copy(dst2.at[pl.ds(dbase, 8)], d0, semD0)
    pltpu.async_copy(dst2.at[pl.ds(dbase + 8, 8)], d1, semD1)
    pltpu.async_copy(r_hbm.at[src_v.at[0]], rows0, semA)
    pltpu.async_copy(r_hbm.at[src_v.at[1]], rows1, semB)

    def superblock(sb, carry):
        for half in range(2):
            b = 2 * sb + half
            di, semD = (d0, semD0) if half == 0 else (d1, semD1)
            pltpu.make_async_copy(dst2.at[pl.ds(dbase + 8 * b, 8)], di, semD).wait()

            def chunkpair(t, carry2):
                for q in range(2):
                    ch = 2 * t + q
                    jj = 8 * b + ch
                    buf, sem = (rows0, semA) if q == 0 else (rows1, semB)
                    pltpu.make_async_copy(r_hbm.at[src_v.at[jj]], buf, sem).wait()
                    pltpu.sync_copy(buf, acc_sh.at[di.at[ch]], add=True)

                    @pl.when(jj + 2 < NCH)
                    def _():
                        pltpu.async_copy(r_hbm.at[src_v.at[jj + 2]], buf, sem)
                return carry2

            lax.fori_loop(0, 4, chunkpair, 0)

            @pl.when(b + 2 < NBLK)
            def _():
                pltpu.async_copy(dst2.at[pl.ds(dbase + 8 * (b + 2), 8)], di, semD)
        return carry

    lax.fori_loop(0, NBLK // 2, superblock, 0)

    plsc.subcore_barrier()
    base = c * NPAD + s * RPT
    pltpu.sync_copy(acc_sh.at[pl.ds(s * RPT, RPT)], out_hbm.at[pl.ds(base, RPT)])


_segsum = pl.kernel(
    _segsum_body,
    out_type=jax.ShapeDtypeStruct((NC * NPAD, D), jnp.float32),
    mesh=_MESH,
    scratch_types=(
        pltpu.VMEM((NCH, K), jnp.int32),
        pltpu.VMEM((8, K), jnp.int32),
        pltpu.VMEM((8, K), jnp.int32),
        pltpu.VMEM((K, D), jnp.float32),
        pltpu.VMEM((K, D), jnp.float32),
        pltpu.VMEM((8, D), jnp.float32),
        pltpu.SemaphoreType.DMA,
        pltpu.SemaphoreType.DMA,
        pltpu.SemaphoreType.DMA,
        pltpu.SemaphoreType.DMA,
        pltpu.VMEM_SHARED((NPAD, D), jnp.float32),
    ),
)


def _deg_body(dstg, z1d, deg_hbm, dst_v, deg_v):
    c = lax.axis_index("c")
    s = lax.axis_index("s")
    w = c * NS + s
    pltpu.sync_copy(z1d, deg_v)
    pltpu.sync_copy(dstg.at[w], dst_v)
    ones16 = jnp.ones((16,), jnp.float32)

    def row(i, carry):
        def col(j, carry2):
            v = dst_v[i, pl.ds(j * 16, 16)]
            plsc.addupdate_scatter(deg_v, [v], ones16)
            return carry2
        return lax.fori_loop(0, K // 16, col, carry)

    lax.fori_loop(0, NCH, row, 0)
    pltpu.sync_copy(deg_v, deg_hbm.at[w])


_deg = pl.kernel(
    _deg_body,
    out_type=jax.ShapeDtypeStruct((NW, NPAD), jnp.float32),
    mesh=_MESH,
    compiler_params=pltpu.CompilerParams(needs_layout_passes=False),
    scratch_types=(
        pltpu.VMEM((NCH, K), jnp.int32),
        pltpu.VMEM((NPAD,), jnp.float32),
    ),
)

_NB = 10
_RB = N_NODES // _NB


def _relu_body(x_ref, o_ref):
    o_ref[...] = jnp.maximum(x_ref[...], 0.0)


_relu = pl.pallas_call(
    _relu_body,
    grid=(_NB,),
    in_specs=[pl.BlockSpec((_RB, D), lambda i: (i, 0))],
    out_specs=pl.BlockSpec((_RB, D), lambda i: (i, 0)),
    out_shape=jax.ShapeDtypeStruct((N_NODES, D), jnp.float32),
)


def _make_combine(apply_relu):
    def body(p_ref, deg_ref, xin_ref, wl_ref, wr_ref, b_ref, o_ref):
        cnt = jnp.maximum(jnp.sum(deg_ref[...], axis=1), 1.0)[:, None]
        agg = (p_ref[0] + p_ref[1]) / cnt
        z = (jnp.dot(agg, wl_ref[...], preferred_element_type=jnp.float32)
             + jnp.dot(xin_ref[...], wr_ref[...], preferred_element_type=jnp.float32)
             + b_ref[...])
        o_ref[...] = jnp.maximum(z, 0.0) if apply_relu else z

    return pl.pallas_call(
        body,
        grid=(_NB,),
        in_specs=[
            pl.BlockSpec((NC, _RB, D), lambda i: (0, i, 0)),
            pl.BlockSpec((_RB, NW), lambda i: (i, 0)),
            pl.BlockSpec((_RB, D), lambda i: (i, 0)),
            pl.BlockSpec((D, D), lambda i: (0, 0)),
            pl.BlockSpec((D, D), lambda i: (0, 0)),
            pl.BlockSpec((1, D), lambda i: (0, 0)),
        ],
        out_specs=pl.BlockSpec((_RB, D), lambda i: (i, 0)),
        out_shape=jax.ShapeDtypeStruct((N_NODES, D), jnp.float32),
    )


_combine_relu = _make_combine(True)
_combine_id = _make_combine(False)


def kernel(x, edge_index, W_l0, b_l0, W_r0, W_l1, b_l1, W_r1):
    ei = edge_index.astype(jnp.int32)
    npad_e = E_PAD - N_EDGES
    src = jnp.concatenate([ei[0], jnp.zeros((npad_e,), jnp.int32)]).reshape(NW, NCH, K)
    dst = jnp.concatenate([ei[1], jnp.full((npad_e,), N_NODES, jnp.int32)]).reshape(NW, NCH, K)
    dst2 = dst.reshape(NW * NCH, K)
    z1d = jnp.zeros((NPAD,), jnp.float32)

    r0 = _relu(x)
    degp = _deg(dst, z1d).T
    p0_flat = _segsum(r0, src, dst2)
    p0 = p0_flat.reshape(NC, NPAD, D)
    z1 = _combine_relu(p0, degp, x, W_l0, W_r0, b_l0.reshape(1, D))
    p1_flat = _segsum(z1, src, dst2)
    p1 = p1_flat.reshape(NC, NPAD, D)
    return _combine_id(p1, degp, z1, W_l1, W_r1, b_l1.reshape(1, D))


# speedup vs baseline: 3.5488x; 3.5488x over previous
import jax
import jax.numpy as jnp
from jax import lax
from jax.experimental import pallas as pl
from jax.experimental.pallas import tpu as pltpu
from jax.experimental.pallas import tpu_sc as plsc

N_NODES = 10000
N_EDGES = 320000
D = 128
NC, NS = 2, 16
NW = NC * NS
NPAD = 10240
RPT = NPAD // NS
K = 128
NCH = 80
NBLK = 10
EPW = NCH * K
E_PAD = NW * EPW

_MESH = plsc.VectorSubcoreMesh(
    core_axis_name="c", subcore_axis_name="s", num_cores=NC, num_subcores=NS)


def _segsum_body(r_hbm, srcg, dst2, out_hbm,
                 src_v, d0, d1, rows0, rows1, zbuf,
                 semA, semB, semD0, semD1, acc_sh):
    c = lax.axis_index("c")
    s = lax.axis_index("s")
    w = c * NS + s
    zv = jnp.zeros((16,), jnp.float32)
    for r in range(8):
        for k in range(D // 16):
            zbuf[r, pl.ds(k * 16, 16)] = zv

    def zrow(i, carry):
        pltpu.sync_copy(zbuf, acc_sh.at[pl.ds(s * RPT + i * 8, 8)])
        return carry

    lax.fori_loop(0, RPT // 8, zrow, 0)
    pltpu.sync_copy(srcg.at[w], src_v)
    plsc.subcore_barrier()

    dbase = w * NCH
    pltpu.async_copy(dst2.at[pl.ds(dbase, 8)], d0, semD0)
    pltpu.async_copy(dst2.at[pl.ds(dbase + 8, 8)], d1, semD1)
    pltpu.async_copy(r_hbm.at[src_v.at[0]], rows0, semA)
    pltpu.async_copy(r_hbm.at[src_v.at[1]], rows1, semB)

    def superblock(sb, carry):
        for half in range(2):
            b = 2 * sb + half
            di, semD = (d0, semD0) if half == 0 else (d1, semD1)
            pltpu.make_async_copy(dst2.at[pl.ds(dbase + 8 * b, 8)], di, semD).wait()

            def chunkpair(t, carry2):
                for q in range(2):
                    ch = 2 * t + q
                    jj = 8 * b + ch
                    buf, sem = (rows0, semA) if q == 0 else (rows1, semB)
                    pltpu.make_async_copy(r_hbm.at[src_v.at[jj]], buf, sem).wait()
                    pltpu.sync_copy(buf, acc_sh.at[di.at[ch]], add=True)

                    @pl.when(jj + 2 < NCH)
                    def _():
                        pltpu.async_copy(r_hbm.at[src_v.at[jj + 2]], buf, sem)
                return carry2

            lax.fori_loop(0, 4, chunkpair, 0)

            @pl.when(b + 2 < NBLK)
            def _():
                pltpu.async_copy(dst2.at[pl.ds(dbase + 8 * (b + 2), 8)], di, semD)
        return carry

    lax.fori_loop(0, NBLK // 2, superblock, 0)

    plsc.subcore_barrier()
    base = c * NPAD + s * RPT
    pltpu.sync_copy(acc_sh.at[pl.ds(s * RPT, RPT)], out_hbm.at[pl.ds(base, RPT)])


_segsum = pl.kernel(
    _segsum_body,
    out_type=jax.ShapeDtypeStruct((NC * NPAD, D), jnp.float32),
    mesh=_MESH,
    scratch_types=(
        pltpu.VMEM((NCH, K), jnp.int32),
        pltpu.VMEM((8, K), jnp.int32),
        pltpu.VMEM((8, K), jnp.int32),
        pltpu.VMEM((K, D), jnp.float32),
        pltpu.VMEM((K, D), jnp.float32),
        pltpu.VMEM((8, D), jnp.float32),
        pltpu.SemaphoreType.DMA,
        pltpu.SemaphoreType.DMA,
        pltpu.SemaphoreType.DMA,
        pltpu.SemaphoreType.DMA,
        pltpu.VMEM_SHARED((NPAD, D), jnp.float32),
    ),
)


def _deg_body(dstg, z1d, deg_hbm, dst_v, deg_v):
    c = lax.axis_index("c")
    s = lax.axis_index("s")
    w = c * NS + s
    pltpu.sync_copy(z1d, deg_v)
    pltpu.sync_copy(dstg.at[w], dst_v)
    ones16 = jnp.ones((16,), jnp.float32)

    def row(i, carry):
        def col(j, carry2):
            v = dst_v[i, pl.ds(j * 16, 16)]
            plsc.addupdate_scatter(deg_v, [v], ones16)
            return carry2
        return lax.fori_loop(0, K // 16, col, carry)

    lax.fori_loop(0, NCH, row, 0)
    pltpu.sync_copy(deg_v, deg_hbm.at[w])


_deg = pl.kernel(
    _deg_body,
    out_type=jax.ShapeDtypeStruct((NW, NPAD), jnp.float32),
    mesh=_MESH,
    compiler_params=pltpu.CompilerParams(needs_layout_passes=False),
    scratch_types=(
        pltpu.VMEM((NCH, K), jnp.int32),
        pltpu.VMEM((NPAD,), jnp.float32),
    ),
)

_NB = 10
_RB = N_NODES // _NB


def _relu_body(x_ref, o_ref):
    o_ref[...] = jnp.maximum(x_ref[...], 0.0)


_relu = pl.pallas_call(
    _relu_body,
    grid=(_NB,),
    in_specs=[pl.BlockSpec((_RB, D), lambda i: (i, 0))],
    out_specs=pl.BlockSpec((_RB, D), lambda i: (i, 0)),
    out_shape=jax.ShapeDtypeStruct((N_NODES, D), jnp.float32),
)


def _make_combine(apply_relu):
    def body(p_ref, deg_ref, xin_ref, wl_ref, wr_ref, b_ref, o_ref):
        cnt = jnp.maximum(jnp.sum(deg_ref[...], axis=1), 1.0)[:, None]
        agg = (p_ref[0] + p_ref[1]) / cnt
        z = (jnp.dot(agg, wl_ref[...], preferred_element_type=jnp.float32)
             + jnp.dot(xin_ref[...], wr_ref[...], preferred_element_type=jnp.float32)
             + b_ref[...])
        o_ref[...] = jnp.maximum(z, 0.0) if apply_relu else z

    return pl.pallas_call(
        body,
        grid=(_NB,),
        in_specs=[
            pl.BlockSpec((NC, _RB, D), lambda i: (0, i, 0)),
            pl.BlockSpec((_RB, NW), lambda i: (i, 0)),
            pl.BlockSpec((_RB, D), lambda i: (i, 0)),
            pl.BlockSpec((D, D), lambda i: (0, 0)),
            pl.BlockSpec((D, D), lambda i: (0, 0)),
            pl.BlockSpec((1, D), lambda i: (0, 0)),
        ],
        out_specs=pl.BlockSpec((_RB, D), lambda i: (i, 0)),
        out_shape=jax.ShapeDtypeStruct((N_NODES, D), jnp.float32),
    )


_combine_relu = _make_combine(True)
_combine_id = _make_combine(False)


def kernel(x, edge_index, W_l0, b_l0, W_r0, W_l1, b_l1, W_r1):
    ei = edge_index.astype(jnp.int32)
    npad_e = E_PAD - N_EDGES
    pad_iota = lax.iota(jnp.int32, npad_e)
    src = jnp.concatenate([ei[0], pad_iota % N_NODES]).reshape(NW, NCH, K)
    dst = jnp.concatenate([ei[1], N_NODES + pad_iota % (NPAD - N_NODES)]).reshape(NW, NCH, K)
    dst2 = dst.reshape(NW * NCH, K)
    z1d = jnp.zeros((NPAD,), jnp.float32)

    r0 = _relu(x)
    degp = _deg(dst, z1d).T
    p0_flat = _segsum(r0, src, dst2)
    p0 = p0_flat.reshape(NC, NPAD, D)
    z1 = _combine_relu(p0, degp, x, W_l0, W_r0, b_l0.reshape(1, D))
    p1_flat = _segsum(z1, src, dst2)
    p1 = p1_flat.reshape(NC, NPAD, D)
    return _combine_id(p1, degp, z1, W_l1, W_r1, b_l1.reshape(1, D))


# speedup vs baseline: 3.6793x; 1.0368x over previous
import jax
import jax.numpy as jnp
from jax import lax
from jax.experimental import pallas as pl
from jax.experimental.pallas import tpu as pltpu
from jax.experimental.pallas import tpu_sc as plsc

N_NODES = 10000
N_EDGES = 320000
D = 128
NC, NS = 2, 16
NW = NC * NS
NPAD = 10240
RPT = NPAD // NS
K = 128
NCH = 80
NBLK = 10
EPW = NCH * K
E_PAD = NW * EPW

_MESH = plsc.VectorSubcoreMesh(
    core_axis_name="c", subcore_axis_name="s", num_cores=NC, num_subcores=NS)


def _segsum_body(r_hbm, srcg, dst2, out_hbm,
                 src_v, d0, d1, rows0, rows1, zbuf,
                 semA, semB, semD0, semD1, semZ, acc_sh):
    c = lax.axis_index("c")
    s = lax.axis_index("s")
    w = c * NS + s
    zv = jnp.zeros((16,), jnp.float32)
    for r in range(8):
        for k in range(D // 16):
            zbuf[r, pl.ds(k * 16, 16)] = zv

    def zrow(i, carry):
        pltpu.async_copy(zbuf, acc_sh.at[pl.ds(s * RPT + i * 8, 8)], semZ)
        return carry

    lax.fori_loop(0, RPT // 8, zrow, 0)
    pltpu.sync_copy(srcg.at[w], src_v)

    def zdrain(i, carry):
        pltpu.make_async_copy(zbuf, acc_sh.at[pl.ds(s * RPT + i * 8, 8)], semZ).wait()
        return carry

    lax.fori_loop(0, RPT // 8, zdrain, 0)
    plsc.subcore_barrier()

    dbase = w * NCH
    pltpu.async_copy(dst2.at[pl.ds(dbase, 8)], d0, semD0)
    pltpu.async_copy(dst2.at[pl.ds(dbase + 8, 8)], d1, semD1)
    pltpu.async_copy(r_hbm.at[src_v.at[0]], rows0, semA)
    pltpu.async_copy(r_hbm.at[src_v.at[1]], rows1, semB)

    def superblock(sb, carry):
        for half in range(2):
            b = 2 * sb + half
            di, semD = (d0, semD0) if half == 0 else (d1, semD1)
            pltpu.make_async_copy(dst2.at[pl.ds(dbase + 8 * b, 8)], di, semD).wait()

            def chunkpair(t, carry2):
                for q in range(2):
                    ch = 2 * t + q
                    jj = 8 * b + ch
                    buf, sem = (rows0, semA) if q == 0 else (rows1, semB)
                    pltpu.make_async_copy(r_hbm.at[src_v.at[jj]], buf, sem).wait()
                    pltpu.sync_copy(buf, acc_sh.at[di.at[ch]], add=True)

                    @pl.when(jj + 2 < NCH)
                    def _():
                        pltpu.async_copy(r_hbm.at[src_v.at[jj + 2]], buf, sem)
                return carry2

            lax.fori_loop(0, 4, chunkpair, 0)

            @pl.when(b + 2 < NBLK)
            def _():
                pltpu.async_copy(dst2.at[pl.ds(dbase + 8 * (b + 2), 8)], di, semD)
        return carry

    lax.fori_loop(0, NBLK // 2, superblock, 0)

    plsc.subcore_barrier()
    base = c * NPAD + s * RPT
    pltpu.sync_copy(acc_sh.at[pl.ds(s * RPT, RPT)], out_hbm.at[pl.ds(base, RPT)])


_segsum = pl.kernel(
    _segsum_body,
    out_type=jax.ShapeDtypeStruct((NC * NPAD, D), jnp.float32),
    mesh=_MESH,
    scratch_types=(
        pltpu.VMEM((NCH, K), jnp.int32),
        pltpu.VMEM((8, K), jnp.int32),
        pltpu.VMEM((8, K), jnp.int32),
        pltpu.VMEM((K, D), jnp.float32),
        pltpu.VMEM((K, D), jnp.float32),
        pltpu.VMEM((8, D), jnp.float32),
        pltpu.SemaphoreType.DMA,
        pltpu.SemaphoreType.DMA,
        pltpu.SemaphoreType.DMA,
        pltpu.SemaphoreType.DMA,
        pltpu.SemaphoreType.DMA,
        pltpu.VMEM_SHARED((NPAD, D), jnp.float32),
    ),
)


def _deg_body(dstg, z1d, deg_hbm, dst_v, deg_v):
    c = lax.axis_index("c")
    s = lax.axis_index("s")
    w = c * NS + s
    pltpu.sync_copy(z1d, deg_v)
    pltpu.sync_copy(dstg.at[w], dst_v)
    ones16 = jnp.ones((16,), jnp.float32)

    def row(i, carry):
        def col(j, carry2):
            v = dst_v[i, pl.ds(j * 16, 16)]
            plsc.addupdate_scatter(deg_v, [v], ones16)
            return carry2
        return lax.fori_loop(0, K // 16, col, carry)

    lax.fori_loop(0, NCH, row, 0)
    pltpu.sync_copy(deg_v, deg_hbm.at[w])


_deg = pl.kernel(
    _deg_body,
    out_type=jax.ShapeDtypeStruct((NW, NPAD), jnp.float32),
    mesh=_MESH,
    compiler_params=pltpu.CompilerParams(needs_layout_passes=False),
    scratch_types=(
        pltpu.VMEM((NCH, K), jnp.int32),
        pltpu.VMEM((NPAD,), jnp.float32),
    ),
)

_NB = 10
_RB = N_NODES // _NB


def _relu_body(x_ref, o_ref):
    o_ref[...] = jnp.maximum(x_ref[...], 0.0)


_relu = pl.pallas_call(
    _relu_body,
    grid=(_NB,),
    in_specs=[pl.BlockSpec((_RB, D), lambda i: (i, 0))],
    out_specs=pl.BlockSpec((_RB, D), lambda i: (i, 0)),
    out_shape=jax.ShapeDtypeStruct((N_NODES, D), jnp.float32),
)


def _make_combine(apply_relu):
    def body(p_ref, deg_ref, xin_ref, wl_ref, wr_ref, b_ref, o_ref):
        cnt = jnp.maximum(jnp.sum(deg_ref[...], axis=1), 1.0)[:, None]
        agg = (p_ref[0] + p_ref[1]) / cnt
        z = (jnp.dot(agg, wl_ref[...], preferred_element_type=jnp.float32)
             + jnp.dot(xin_ref[...], wr_ref[...], preferred_element_type=jnp.float32)
             + b_ref[...])
        o_ref[...] = jnp.maximum(z, 0.0) if apply_relu else z

    return pl.pallas_call(
        body,
        grid=(_NB,),
        in_specs=[
            pl.BlockSpec((NC, _RB, D), lambda i: (0, i, 0)),
            pl.BlockSpec((_RB, NW), lambda i: (i, 0)),
            pl.BlockSpec((_RB, D), lambda i: (i, 0)),
            pl.BlockSpec((D, D), lambda i: (0, 0)),
            pl.BlockSpec((D, D), lambda i: (0, 0)),
            pl.BlockSpec((1, D), lambda i: (0, 0)),
        ],
        out_specs=pl.BlockSpec((_RB, D), lambda i: (i, 0)),
        out_shape=jax.ShapeDtypeStruct((N_NODES, D), jnp.float32),
    )


_combine_relu = _make_combine(True)
_combine_id = _make_combine(False)


def kernel(x, edge_index, W_l0, b_l0, W_r0, W_l1, b_l1, W_r1):
    ei = edge_index.astype(jnp.int32)
    npad_e = E_PAD - N_EDGES
    pad_iota = lax.iota(jnp.int32, npad_e)
    src = jnp.concatenate([ei[0], pad_iota % N_NODES]).reshape(NW, NCH, K)
    dst = jnp.concatenate([ei[1], N_NODES + pad_iota % (NPAD - N_NODES)]).reshape(NW, NCH, K)
    dst2 = dst.reshape(NW * NCH, K)
    z1d = jnp.zeros((NPAD,), jnp.float32)

    r0 = _relu(x)
    degp = _deg(dst, z1d).T
    p0_flat = _segsum(r0, src, dst2)
    p0 = p0_flat.reshape(NC, NPAD, D)
    z1 = _combine_relu(p0, degp, x, W_l0, W_r0, b_l0.reshape(1, D))
    p1_flat = _segsum(z1, src, dst2)
    p1 = p1_flat.reshape(NC, NPAD, D)
    return _combine_id(p1, degp, z1, W_l1, W_r1, b_l1.reshape(1, D))
